# baseline (speedup 1.0000x reference)
import functools

import jax
import jax.numpy as jnp
from jax import lax
from jax.experimental import pallas as pl
from jax.experimental.pallas import tpu as pltpu
from jax.experimental.pallas import tpu_sc as plsc

_E = 320000
_TSL = 20000
_H = 10000
_LBH = 625
_BINS = 256
_SH = (0, 8, 16, 24)
_EPW = 10000
_GOCH = 400
_GICH = 80


def _node_table_body(x_ref, m_ref, c_ref, o_ref):
    o_ref[...] = (
        jnp.dot(x_ref[...], m_ref[...], preferred_element_type=jnp.float32)
        + c_ref[...]
    )


def _sort_gather_body(t_hbm, ea_hbm, dst_hbm, src_hbm, tg_hbm, ej_hbm,
                      ka_sh, va_sh, kb_sh, vb_sh, ts_sh,
                      keych, valch, hist, offs, tsall, totv, pv, gpv,
                      kbuf, vbuf, pbuf, kbuf1, vbuf1, pbuf1,
                      kbuf2, vbuf2, pbuf2,
                      permv, dstv, rows, sema, semb):
    cid = lax.axis_index("c")
    sid = lax.axis_index("s")
    tid = sid
    i16 = lax.iota(jnp.int32, 16)
    lbh = i16 * _LBH
    g0 = tid * _TSL

    ones = jnp.ones((16,), jnp.int32)

    def load_half(p, h, src_k, src_v):
        if p == 0:
            pltpu.sync_copy(dst_hbm.at[pl.ds(g0 + h * _H, _H)], keych)
            pltpu.sync_copy(src_hbm.at[pl.ds(g0 + h * _H, _H)], valch)

            def keyinit(v, carry):
                sl = pl.ds(v * 16, 16)
                keych[sl] = keych[sl] * 10000 + valch[sl]
                valch[sl] = g0 + h * _H + v * 16 + i16
                return carry
            lax.fori_loop(0, _H // 16, keyinit, 0)
        else:
            pltpu.sync_copy(src_k.at[pl.ds(g0 + h * _H, _H)], keych)
            pltpu.sync_copy(src_v.at[pl.ds(g0 + h * _H, _H)], valch)

    for p, sh in enumerate(_SH):
        src_k, src_v = (ka_sh, va_sh) if p % 2 == 0 else (kb_sh, vb_sh)
        dst_k, dst_v = (kb_sh, vb_sh) if p % 2 == 0 else (ka_sh, va_sh)

        def zero(i, carry):
            hist[pl.ds(i * 16, 16)] = jnp.zeros((16,), jnp.int32)
            return carry
        lax.fori_loop(0, (_BINS * 32) // 16, zero, 0)

        for h in (0, 1):
            load_half(p, h, src_k, src_v)
            grp = h * 16 + i16

            def histo(v, carry):
                kv = plsc.load_gather(keych, [lbh + v])
                digit = (kv >> sh) & (_BINS - 1)
                plsc.addupdate_scatter(hist, [digit * 32 + grp], ones)
                return carry
            lax.fori_loop(0, _LBH, histo, 0)

        def tsum(dc, carry):
            acc = jnp.zeros((16,), jnp.int32)
            dbase = (dc * 16 + i16) * 32
            for g in range(32):
                acc = acc + plsc.load_gather(hist, [dbase + g])
            totv[pl.ds(dc * 16, 16)] = acc
            return carry
        lax.fori_loop(0, _BINS // 16, tsum, 0)
        pltpu.sync_copy(totv, ts_sh.at[tid])
        plsc.subcore_barrier()

        pltpu.sync_copy(ts_sh, tsall)

        def scan1(dc, carry):
            sl = pl.ds(dc * 16, 16)
            tot = jnp.zeros((16,), jnp.int32)
            pfx = jnp.zeros((16,), jnp.int32)
            for t in range(16):
                v = tsall[t, sl]
                tot = tot + v
                pfx = pfx + v * jnp.where(t < tid, 1, 0).astype(jnp.int32)
            totv[sl] = tot
            pv[sl] = pfx
            return carry
        lax.fori_loop(0, _BINS // 16, scan1, 0)

        def scan2(dc, carry):
            sl = pl.ds(dc * 16, 16)
            ch = totv[sl]
            excl = plsc.cumsum(ch) - ch
            gpv[sl] = excl + carry + pv[sl]
            return carry + jnp.sum(ch)
        lax.fori_loop(0, _BINS // 16, scan2, jnp.int32(0))

        def mkoffs(d, carry):
            h0 = plsc.load_gather(hist, [d * 32 + i16])
            h1 = plsc.load_gather(hist, [d * 32 + 16 + i16])
            base = plsc.load_gather(gpv, [jnp.full((16,), d, jnp.int32)])
            offs[pl.ds(d * 32, 16)] = base + (plsc.cumsum(h0) - h0)
            offs[pl.ds(d * 32 + 16, 16)] = (base + jnp.sum(h0)
                                            + (plsc.cumsum(h1) - h1))
            return carry
        lax.fori_loop(0, _BINS, mkoffs, 0)

        for h in (0, 1):
            load_half(p, h, src_k, src_v)
            grp = h * 16 + i16

            def rank1(v):
                kv = plsc.load_gather(keych, [lbh + v])
                vv = plsc.load_gather(valch, [lbh + v])
                digit = (kv >> sh) & (_BINS - 1)
                b = digit * 32 + grp
                pos = plsc.load_gather(offs, [b])
                plsc.store_scatter(offs, [b], pos + 1)
                return kv, vv, pos

            bufs = ((kbuf, vbuf, pbuf, sema), (kbuf1, vbuf1, pbuf1, semb))

            def rank_chunk(c, kb_, vb_, pb_):
                for u in range(8):
                    kv, vv, pos = rank1(c * 8 + u)
                    usl = pl.ds(u * 16, 16)
                    kb_[usl] = kv
                    vb_[usl] = vv
                    pb_[usl] = pos

            def fire(kb_, vb_, pb_, sem):
                pltpu.async_copy(kb_, dst_k.at[pb_], sem)
                pltpu.async_copy(vb_, dst_v.at[pb_], sem)

            def drain(kb_, vb_, pb_, sem):
                pltpu.make_async_copy(kb_, dst_k.at[pb_], sem).wait()
                pltpu.make_async_copy(vb_, dst_v.at[pb_], sem).wait()

            for s in (0, 1):
                rank_chunk(s, *bufs[s][:3])
                fire(*bufs[s])

            def permute2(c2, carry):
                for s in (0, 1):
                    drain(*bufs[s])
                    rank_chunk(2 + c2 * 2 + s, *bufs[s][:3])
                    fire(*bufs[s])
                return carry
            lax.fori_loop(0, (_LBH // 8 - 2) // 2, permute2, 0)
            for s in (0, 1):
                drain(*bufs[s])

            kv, vv, pos = rank1(_LBH - 1)
            kbuf2[...] = kv
            vbuf2[...] = vv
            pbuf2[...] = pos
            ca = pltpu.async_copy(kbuf2, dst_k.at[pbuf2], sema)
            cb = pltpu.async_copy(vbuf2, dst_v.at[pbuf2], semb)
            ca.wait()
            cb.wait()
        plsc.subcore_barrier()

    perm_sh = va_sh

    wid = sid * 2 + cid
    e0 = wid * _EPW

    def gouter(o, carry):
        ob = e0 + o * _GOCH
        pltpu.sync_copy(dst_hbm.at[pl.ds(ob, _GOCH)], dstv)
        pltpu.sync_copy(perm_sh.at[pl.ds(ob, _GOCH)], permv)
        copies = []
        for j in range(_GOCH // _GICH):
            sl = pl.ds(j * _GICH, _GICH)
            copies.append(pltpu.async_copy(
                t_hbm.at[dstv.at[sl]], rows.at[sl], sema))
        for cc in copies:
            cc.wait()
        pltpu.sync_copy(rows, tg_hbm.at[pl.ds(ob, _GOCH)])
        copies = []
        for j in range(_GOCH // _GICH):
            sl = pl.ds(j * _GICH, _GICH)
            copies.append(pltpu.async_copy(
                ea_hbm.at[permv.at[sl]], rows.at[sl], semb))
        for cc in copies:
            cc.wait()
        pltpu.sync_copy(rows, ej_hbm.at[pl.ds(ob, _GOCH)])
        return carry
    lax.fori_loop(0, _EPW // _GOCH, gouter, 0)


def _dense_body(ea_ref, tg_ref, ej_ref, wa_ref, bw_ref, ew_ref, cons_ref,
                o_ref):
    ea = ea_ref[...]
    h = tg_ref[...] + jnp.dot(ea, wa_ref[...],
                              preferred_element_type=jnp.float32)
    e2 = jnp.dot(ej_ref[...], bw_ref[...],
                 preferred_element_type=jnp.float32) + cons_ref[0:1, :]
    s = jnp.dot(h + e2, ew_ref[...],
                preferred_element_type=jnp.float32) + cons_ref[1:2, :]
    g = jax.nn.sigmoid(s)
    t = g * h * cons_ref[2:3, :] + cons_ref[3:4, :]
    o_ref[...] = ea + jnp.where(t >= 0, t, 0.01 * t)


def kernel(x, edge_index, edge_attr, edge_mask, A_W, A_b, B_W, B_b, C_W, C_b,
           D_W, D_b, E_W, E_b, bn_gamma, bn_beta, bn_mean, bn_var):
    N, NC = x.shape
    E, EC = edge_attr.shape
    del edge_mask

    dst = edge_index[1]
    src = edge_index[0]

    D1 = D_W[:, :NC]
    D2 = D_W[:, NC:]
    M = C_W.T @ D1.T
    c0 = C_b @ D1.T + A_b @ D2.T + D_b
    WA = A_W.T @ D2.T
    scale = bn_gamma * jax.lax.rsqrt(bn_var + 1e-5)
    shift = bn_beta - bn_mean * scale

    t_tab = pl.pallas_call(
        _node_table_body,
        out_shape=jax.ShapeDtypeStruct((N, EC), jnp.float32),
    )(x, M, c0[None, :])

    mesh = plsc.VectorSubcoreMesh(core_axis_name="c", subcore_axis_name="s")
    sort_gather = functools.partial(
        pl.kernel,
        out_type=(jax.ShapeDtypeStruct((E, EC), jnp.float32),
                  jax.ShapeDtypeStruct((E, EC), jnp.float32)),
        mesh=mesh,
        compiler_params=pltpu.CompilerParams(use_tc_tiling_on_sc=False,
                                             needs_layout_passes=False),
        scratch_types=[
            pltpu.VMEM_SHARED((_E,), jnp.int32),
            pltpu.VMEM_SHARED((_E,), jnp.int32),
            pltpu.VMEM_SHARED((_E,), jnp.int32),
            pltpu.VMEM_SHARED((_E,), jnp.int32),
            pltpu.VMEM_SHARED((16, _BINS), jnp.int32),
            pltpu.VMEM((_H,), jnp.int32),
            pltpu.VMEM((_H,), jnp.int32),
            pltpu.VMEM((_BINS * 32,), jnp.int32),
            pltpu.VMEM((_BINS * 32,), jnp.int32),
            pltpu.VMEM((16, _BINS), jnp.int32),
            pltpu.VMEM((_BINS,), jnp.int32),
            pltpu.VMEM((_BINS,), jnp.int32),
            pltpu.VMEM((_BINS,), jnp.int32),
            pltpu.VMEM((128,), jnp.int32),
            pltpu.VMEM((128,), jnp.int32),
            pltpu.VMEM((128,), jnp.int32),
            pltpu.VMEM((128,), jnp.int32),
            pltpu.VMEM((128,), jnp.int32),
            pltpu.VMEM((128,), jnp.int32),
            pltpu.VMEM((16,), jnp.int32),
            pltpu.VMEM((16,), jnp.int32),
            pltpu.VMEM((16,), jnp.int32),
            pltpu.VMEM((_GOCH,), jnp.int32),
            pltpu.VMEM((_GOCH,), jnp.int32),
            pltpu.VMEM((_GOCH, EC), jnp.float32),
            pltpu.SemaphoreType.DMA,
            pltpu.SemaphoreType.DMA,
        ],
    )(_sort_gather_body)
    tg, ej = sort_gather(t_tab, edge_attr, dst, src)

    cons = jnp.stack([B_b, E_b, scale, shift])

    BLK = 4000
    grid = (E // BLK,)
    row_spec = pl.BlockSpec((BLK, EC), lambda i: (i, 0))
    full_spec = pl.BlockSpec((EC, EC), lambda i: (0, 0))
    out = pl.pallas_call(
        _dense_body,
        grid=grid,
        in_specs=[row_spec, row_spec, row_spec, full_spec, full_spec,
                  full_spec, pl.BlockSpec((4, EC), lambda i: (0, 0))],
        out_specs=row_spec,
        out_shape=jax.ShapeDtypeStruct((E, EC), jnp.float32),
    )(edge_attr, tg, ej, WA, B_W.T, E_W.T, cons)

    return out


# speedup vs baseline: 1.6024x; 1.6024x over previous
import functools

import jax
import jax.numpy as jnp
from jax import lax
from jax.experimental import pallas as pl
from jax.experimental.pallas import tpu as pltpu
from jax.experimental.pallas import tpu_sc as plsc

_E = 320000
_TSL = 20000
_H = 10000
_LBH = 625
_BINS = 256
_SH = (0, 8, 16, 24)
_EPW = 10000
_GOCH = 400
_GICH = 80


def _node_table_body(x_ref, m_ref, c_ref, o_ref):
    o_ref[...] = (
        jnp.dot(x_ref[...], m_ref[...], preferred_element_type=jnp.float32)
        + c_ref[...]
    )


def _sort_gather_body(t_hbm, ea_hbm, dst_hbm, src_hbm, tg_hbm, ej_hbm,
                      ka_sh, va_sh, kb_sh, vb_sh, ts_sh,
                      keych, valch, hist, offs, tsall, totv, pv, gpv,
                      kbuf, vbuf, pbuf, kbuf1, vbuf1, pbuf1,
                      kbuf2, vbuf2, pbuf2,
                      permv, dstv, rows, sema, semb):
    cid = lax.axis_index("c")
    sid = lax.axis_index("s")
    tid = sid
    i16 = lax.iota(jnp.int32, 16)
    lbh = i16 * _LBH
    g0 = tid * _TSL

    ones = jnp.ones((16,), jnp.int32)

    def load_half(p, h, src_k, src_v):
        if p == 0:
            pltpu.sync_copy(dst_hbm.at[pl.ds(g0 + h * _H, _H)], keych)
            pltpu.sync_copy(src_hbm.at[pl.ds(g0 + h * _H, _H)], valch)

            def keyinit(v, carry):
                sl = pl.ds(v * 16, 16)
                keych[sl] = keych[sl] * 10000 + valch[sl]
                valch[sl] = g0 + h * _H + v * 16 + i16
                return carry
            lax.fori_loop(0, _H // 16, keyinit, 0)
        else:
            pltpu.sync_copy(src_k.at[pl.ds(g0 + h * _H, _H)], keych)
            pltpu.sync_copy(src_v.at[pl.ds(g0 + h * _H, _H)], valch)

    for p, sh in enumerate(_SH):
        src_k, src_v = (ka_sh, va_sh) if p % 2 == 0 else (kb_sh, vb_sh)
        dst_k, dst_v = (kb_sh, vb_sh) if p % 2 == 0 else (ka_sh, va_sh)

        def zero(i, carry):
            hist[pl.ds(i * 16, 16)] = jnp.zeros((16,), jnp.int32)
            return carry
        lax.fori_loop(0, (_BINS * 32) // 16, zero, 0)

        for h in (0, 1):
            load_half(p, h, src_k, src_v)
            grp = h * 16 + i16

            def histo(v, carry):
                kv = plsc.load_gather(keych, [lbh + v])
                digit = (kv >> sh) & (_BINS - 1)
                plsc.addupdate_scatter(hist, [digit * 32 + grp], ones)
                return carry
            lax.fori_loop(0, _LBH, histo, 0)

        def tsum(dc, carry):
            acc = jnp.zeros((16,), jnp.int32)
            dbase = (dc * 16 + i16) * 32
            for g in range(32):
                acc = acc + plsc.load_gather(hist, [dbase + g])
            totv[pl.ds(dc * 16, 16)] = acc
            return carry
        lax.fori_loop(0, _BINS // 16, tsum, 0)
        pltpu.sync_copy(totv, ts_sh.at[tid])
        plsc.subcore_barrier()

        pltpu.sync_copy(ts_sh, tsall)

        def scan1(dc, carry):
            sl = pl.ds(dc * 16, 16)
            tot = jnp.zeros((16,), jnp.int32)
            pfx = jnp.zeros((16,), jnp.int32)
            for t in range(16):
                v = tsall[t, sl]
                tot = tot + v
                pfx = pfx + v * jnp.where(t < tid, 1, 0).astype(jnp.int32)
            totv[sl] = tot
            pv[sl] = pfx
            return carry
        lax.fori_loop(0, _BINS // 16, scan1, 0)

        def scan2(dc, carry):
            sl = pl.ds(dc * 16, 16)
            ch = totv[sl]
            excl = plsc.cumsum(ch) - ch
            gpv[sl] = excl + carry + pv[sl]
            return carry + jnp.sum(ch)
        lax.fori_loop(0, _BINS // 16, scan2, jnp.int32(0))

        def mkoffs(d, carry):
            h0 = plsc.load_gather(hist, [d * 32 + i16])
            h1 = plsc.load_gather(hist, [d * 32 + 16 + i16])
            base = plsc.load_gather(gpv, [jnp.full((16,), d, jnp.int32)])
            offs[pl.ds(d * 32, 16)] = base + (plsc.cumsum(h0) - h0)
            offs[pl.ds(d * 32 + 16, 16)] = (base + jnp.sum(h0)
                                            + (plsc.cumsum(h1) - h1))
            return carry
        lax.fori_loop(0, _BINS, mkoffs, 0)

        for h in (0, 1):
            load_half(p, h, src_k, src_v)
            grp = h * 16 + i16

            def rank1(v):
                kv = plsc.load_gather(keych, [lbh + v])
                vv = plsc.load_gather(valch, [lbh + v])
                digit = (kv >> sh) & (_BINS - 1)
                b = digit * 32 + grp
                pos = plsc.load_gather(offs, [b])
                plsc.store_scatter(offs, [b], pos + 1)
                return kv, vv, pos

            bufs = ((kbuf, vbuf, pbuf, sema), (kbuf1, vbuf1, pbuf1, semb))

            def rank_chunk(c, kb_, vb_, pb_):
                for u in range(8):
                    kv, vv, pos = rank1(c * 8 + u)
                    usl = pl.ds(u * 16, 16)
                    kb_[usl] = kv
                    vb_[usl] = vv
                    pb_[usl] = pos

            def fire(kb_, vb_, pb_, sem):
                pltpu.async_copy(kb_, dst_k.at[pb_], sem)
                pltpu.async_copy(vb_, dst_v.at[pb_], sem)

            def drain(kb_, vb_, pb_, sem):
                pltpu.make_async_copy(kb_, dst_k.at[pb_], sem).wait()
                pltpu.make_async_copy(vb_, dst_v.at[pb_], sem).wait()

            for s in (0, 1):
                rank_chunk(s, *bufs[s][:3])
                fire(*bufs[s])

            def permute2(c2, carry):
                for s in (0, 1):
                    drain(*bufs[s])
                    rank_chunk(2 + c2 * 2 + s, *bufs[s][:3])
                    fire(*bufs[s])
                return carry
            lax.fori_loop(0, (_LBH // 8 - 2) // 2, permute2, 0)
            for s in (0, 1):
                drain(*bufs[s])

            kv, vv, pos = rank1(_LBH - 1)
            kbuf2[...] = kv
            vbuf2[...] = vv
            pbuf2[...] = pos
            ca = pltpu.async_copy(kbuf2, dst_k.at[pbuf2], sema)
            cb = pltpu.async_copy(vbuf2, dst_v.at[pbuf2], semb)
            ca.wait()
            cb.wait()
        plsc.subcore_barrier()

    perm_sh = va_sh

    wid = sid * 2 + cid
    e0 = wid * _EPW

    def gouter(o, carry):
        ob = e0 + o * _GOCH
        pltpu.sync_copy(dst_hbm.at[pl.ds(ob, _GOCH)], dstv)
        pltpu.sync_copy(perm_sh.at[pl.ds(ob, _GOCH)], permv)
        copies = []
        for j in range(_GOCH // _GICH):
            sl = pl.ds(j * _GICH, _GICH)
            copies.append(pltpu.async_copy(
                t_hbm.at[dstv.at[sl]], rows.at[sl], sema))
        for cc in copies:
            cc.wait()
        pltpu.sync_copy(rows, tg_hbm.at[pl.ds(ob, _GOCH)])
        copies = []
        for j in range(_GOCH // _GICH):
            sl = pl.ds(j * _GICH, _GICH)
            copies.append(pltpu.async_copy(
                ea_hbm.at[permv.at[sl]], rows.at[sl], semb))
        for cc in copies:
            cc.wait()
        pltpu.sync_copy(rows, ej_hbm.at[pl.ds(ob, _GOCH)])
        return carry
    lax.fori_loop(0, _EPW // _GOCH, gouter, 0)


def _dense_body(ea_ref, tg_ref, ej_ref, wa_ref, bw_ref, ew_ref, cons_ref,
                o_ref):
    ea = ea_ref[...]
    h = tg_ref[...] + jnp.dot(ea, wa_ref[...],
                              preferred_element_type=jnp.float32)
    e2 = jnp.dot(ej_ref[...], bw_ref[...],
                 preferred_element_type=jnp.float32) + cons_ref[0:1, :]
    s = jnp.dot(h + e2, ew_ref[...],
                preferred_element_type=jnp.float32) + cons_ref[1:2, :]
    g = jax.nn.sigmoid(s)
    t = g * h * cons_ref[2:3, :] + cons_ref[3:4, :]
    o_ref[...] = ea + jnp.where(t >= 0, t, 0.01 * t)


def kernel(x, edge_index, edge_attr, edge_mask, A_W, A_b, B_W, B_b, C_W, C_b,
           D_W, D_b, E_W, E_b, bn_gamma, bn_beta, bn_mean, bn_var):
    N, NC = x.shape
    E, EC = edge_attr.shape
    del edge_mask

    dst = edge_index[1]
    src = edge_index[0]

    D1 = D_W[:, :NC]
    D2 = D_W[:, NC:]
    M = C_W.T @ D1.T
    c0 = C_b @ D1.T + A_b @ D2.T + D_b
    WA = A_W.T @ D2.T
    scale = bn_gamma * jax.lax.rsqrt(bn_var + 1e-5)
    shift = bn_beta - bn_mean * scale

    t_tab = pl.pallas_call(
        _node_table_body,
        out_shape=jax.ShapeDtypeStruct((N, EC), jnp.float32),
    )(x, M, c0[None, :])

    mesh = plsc.VectorSubcoreMesh(core_axis_name="c", subcore_axis_name="s")
    sort_gather = functools.partial(
        pl.kernel,
        out_type=(jax.ShapeDtypeStruct((E, EC), jnp.float32),
                  jax.ShapeDtypeStruct((E, EC), jnp.float32)),
        mesh=mesh,
        compiler_params=pltpu.CompilerParams(use_tc_tiling_on_sc=False,
                                             needs_layout_passes=False),
        scratch_types=[
            pltpu.VMEM_SHARED((_E,), jnp.int32),
            pltpu.VMEM_SHARED((_E,), jnp.int32),
            pltpu.VMEM_SHARED((_E,), jnp.int32),
            pltpu.VMEM_SHARED((_E,), jnp.int32),
            pltpu.VMEM_SHARED((16, _BINS), jnp.int32),
            pltpu.VMEM((_H,), jnp.int32),
            pltpu.VMEM((_H,), jnp.int32),
            pltpu.VMEM((_BINS * 32,), jnp.int32),
            pltpu.VMEM((_BINS * 32,), jnp.int32),
            pltpu.VMEM((16, _BINS), jnp.int32),
            pltpu.VMEM((_BINS,), jnp.int32),
            pltpu.VMEM((_BINS,), jnp.int32),
            pltpu.VMEM((_BINS,), jnp.int32),
            pltpu.VMEM((128,), jnp.int32),
            pltpu.VMEM((128,), jnp.int32),
            pltpu.VMEM((128,), jnp.int32),
            pltpu.VMEM((128,), jnp.int32),
            pltpu.VMEM((128,), jnp.int32),
            pltpu.VMEM((128,), jnp.int32),
            pltpu.VMEM((16,), jnp.int32),
            pltpu.VMEM((16,), jnp.int32),
            pltpu.VMEM((16,), jnp.int32),
            pltpu.VMEM((_GOCH,), jnp.int32),
            pltpu.VMEM((_GOCH,), jnp.int32),
            pltpu.VMEM((_GOCH, EC), jnp.float32),
            pltpu.SemaphoreType.DMA,
            pltpu.SemaphoreType.DMA,
        ],
    )(_sort_gather_body)
    tg, ej = sort_gather(t_tab, edge_attr, dst, src)

    P = 8 * EC
    R = E // 8
    eye8 = jnp.eye(8, dtype=jnp.float32)
    wa_k = jnp.kron(eye8, WA)
    bw_k = jnp.kron(eye8, B_W.T)
    ew_k = jnp.kron(eye8, E_W.T)
    cons = jnp.stack([
        jnp.tile(B_b, 8), jnp.tile(E_b, 8),
        jnp.tile(scale, 8), jnp.tile(shift, 8)])

    BLK = 2000
    grid = (R // BLK,)
    row_spec = pl.BlockSpec((BLK, P), lambda i: (i, 0))
    full_spec = pl.BlockSpec((P, P), lambda i: (0, 0))
    out = pl.pallas_call(
        _dense_body,
        grid=grid,
        in_specs=[row_spec, row_spec, row_spec, full_spec, full_spec,
                  full_spec, pl.BlockSpec((4, P), lambda i: (0, 0))],
        out_specs=row_spec,
        out_shape=jax.ShapeDtypeStruct((R, P), jnp.float32),
    )(edge_attr.reshape(R, P), tg.reshape(R, P), ej.reshape(R, P),
      wa_k, bw_k, ew_k, cons)

    return out.reshape(E, EC)


# speedup vs baseline: 1.8135x; 1.1318x over previous
import functools

import jax
import jax.numpy as jnp
from jax import lax
from jax.experimental import pallas as pl
from jax.experimental.pallas import tpu as pltpu
from jax.experimental.pallas import tpu_sc as plsc

_E = 320000
_TSL = 20000
_H = 10000
_LBH = 625
_BINS = 256
_SH = (0, 8, 16, 24)
_EPW = 10000
_GOCH = 2000
_GICH = 80


def _node_table_body(x_ref, m_ref, c_ref, o_ref):
    o_ref[...] = (
        jnp.dot(x_ref[...], m_ref[...], preferred_element_type=jnp.float32)
        + c_ref[...]
    )


def _sort_body(dst_hbm, src_hbm, perm_hbm,
               ka_sh, va_sh, kb_sh, vb_sh, ts_sh,
               keych, valch, hist, offs, tsall, totv, pv, gpv,
               kbuf0, vbuf0, pbuf0, kbuf1, vbuf1, pbuf1,
               kbuf2, vbuf2, pbuf2, sema, semb):
    sid = lax.axis_index("s")
    tid = sid
    i16 = lax.iota(jnp.int32, 16)
    lbh = i16 * _LBH
    g0 = tid * _TSL

    def load_half(p, h, src_k, src_v):
        if p == 0:
            pltpu.sync_copy(dst_hbm.at[pl.ds(g0 + h * _H, _H)], keych)
            pltpu.sync_copy(src_hbm.at[pl.ds(g0 + h * _H, _H)], valch)

            def keyinit(v, carry):
                sl = pl.ds(v * 16, 16)
                keych[sl] = keych[sl] * 10000 + valch[sl]
                valch[sl] = g0 + h * _H + v * 16 + i16
                return carry
            lax.fori_loop(0, _H // 16, keyinit, 0)
        else:
            pltpu.sync_copy(src_k.at[pl.ds(g0 + h * _H, _H)], keych)
            pltpu.sync_copy(src_v.at[pl.ds(g0 + h * _H, _H)], valch)

    ones = jnp.ones((16,), jnp.int32)
    for p, sh in enumerate(_SH):
        if p == 0:
            src_k = src_v = None
        elif p % 2 == 1:
            src_k, src_v = ka_sh, va_sh
        else:
            src_k, src_v = kb_sh, vb_sh
        dst_k, dst_v = (ka_sh, va_sh) if p % 2 == 0 else (kb_sh, vb_sh)

        def zero(i, carry):
            hist[pl.ds(i * 16, 16)] = jnp.zeros((16,), jnp.int32)
            return carry
        lax.fori_loop(0, (_BINS * 32) // 16, zero, 0)

        for h in (0, 1):
            load_half(p, h, src_k, src_v)
            grp = h * 16 + i16

            def histo(v, carry):
                kv = plsc.load_gather(keych, [lbh + v])
                digit = (kv >> sh) & (_BINS - 1)
                plsc.addupdate_scatter(hist, [digit * 32 + grp], ones)
                return carry
            lax.fori_loop(0, _LBH, histo, 0)

        def tsum(dc, carry):
            acc = jnp.zeros((16,), jnp.int32)
            dbase = (dc * 16 + i16) * 32
            for g in range(32):
                acc = acc + plsc.load_gather(hist, [dbase + g])
            totv[pl.ds(dc * 16, 16)] = acc
            return carry
        lax.fori_loop(0, _BINS // 16, tsum, 0)
        pltpu.sync_copy(totv, ts_sh.at[tid])
        plsc.subcore_barrier()

        pltpu.sync_copy(ts_sh, tsall)

        def scan1(dc, carry):
            sl = pl.ds(dc * 16, 16)
            tot = jnp.zeros((16,), jnp.int32)
            pfx = jnp.zeros((16,), jnp.int32)
            for t in range(16):
                v = tsall[t, sl]
                tot = tot + v
                pfx = pfx + v * jnp.where(t < tid, 1, 0).astype(jnp.int32)
            totv[sl] = tot
            pv[sl] = pfx
            return carry
        lax.fori_loop(0, _BINS // 16, scan1, 0)

        def scan2(dc, carry):
            sl = pl.ds(dc * 16, 16)
            ch = totv[sl]
            excl = plsc.cumsum(ch) - ch
            gpv[sl] = excl + carry + pv[sl]
            return carry + jnp.sum(ch)
        lax.fori_loop(0, _BINS // 16, scan2, jnp.int32(0))

        def mkoffs(d, carry):
            h0 = plsc.load_gather(hist, [d * 32 + i16])
            h1 = plsc.load_gather(hist, [d * 32 + 16 + i16])
            base = plsc.load_gather(gpv, [jnp.full((16,), d, jnp.int32)])
            offs[pl.ds(d * 32, 16)] = base + (plsc.cumsum(h0) - h0)
            offs[pl.ds(d * 32 + 16, 16)] = (base + jnp.sum(h0)
                                            + (plsc.cumsum(h1) - h1))
            return carry
        lax.fori_loop(0, _BINS, mkoffs, 0)

        for h in (0, 1):
            load_half(p, h, src_k, src_v)
            grp = h * 16 + i16

            def rank1(v):
                idx = lbh + v
                kv = plsc.load_gather(keych, [idx])
                vv = plsc.load_gather(valch, [idx])
                digit = (kv >> sh) & (_BINS - 1)
                b = digit * 32 + grp
                pos = plsc.load_gather(offs, [b])
                plsc.store_scatter(offs, [b], pos + 1)
                return kv, vv, pos

            bufs = ((kbuf0, vbuf0, pbuf0, sema), (kbuf1, vbuf1, pbuf1, semb))

            def rank_chunk(c, kb_, vb_, pb_):
                for u in range(8):
                    kv, vv, pos = rank1(c * 8 + u)
                    usl = pl.ds(u * 16, 16)
                    kb_[usl] = kv
                    vb_[usl] = vv
                    pb_[usl] = pos

            def fire(kb_, vb_, pb_, sem):
                pltpu.async_copy(kb_, dst_k.at[pb_], sem)
                pltpu.async_copy(vb_, dst_v.at[pb_], sem)

            def drain(kb_, vb_, pb_, sem):
                pltpu.make_async_copy(kb_, dst_k.at[pb_], sem).wait()
                pltpu.make_async_copy(vb_, dst_v.at[pb_], sem).wait()

            for s in (0, 1):
                rank_chunk(s, *bufs[s][:3])
                fire(*bufs[s])

            def permute2(c2, carry):
                for s in (0, 1):
                    drain(*bufs[s])
                    rank_chunk(2 + c2 * 2 + s, *bufs[s][:3])
                    fire(*bufs[s])
                return carry
            lax.fori_loop(0, (_LBH // 8 - 2) // 2, permute2, 0)
            for s in (0, 1):
                drain(*bufs[s])

            kv, vv, pos = rank1(_LBH - 1)
            kbuf2[...] = kv
            vbuf2[...] = vv
            pbuf2[...] = pos
            ca = pltpu.async_copy(kbuf2, dst_k.at[pbuf2], sema)
            cb = pltpu.async_copy(vbuf2, dst_v.at[pbuf2], semb)
            ca.wait()
            cb.wait()
        plsc.subcore_barrier()

    pltpu.sync_copy(vb_sh.at[pl.ds(g0, _TSL)],
                    perm_hbm.at[pl.ds(g0, _TSL)])


def _gather_body(t_hbm, ea_hbm, dst_hbm, perm_hbm, tg_hbm, ej_hbm,
                 dstv, permv, tgv, ejv, sem_a, sem_b):
    wid = lax.axis_index("s") * 2 + lax.axis_index("c")
    base = wid * _EPW

    def body(o, carry):
        ob = base + o * _GOCH
        pltpu.sync_copy(dst_hbm.at[pl.ds(ob, _GOCH)], dstv)
        pltpu.sync_copy(perm_hbm.at[pl.ds(ob, _GOCH)], permv)
        copies = []
        for j in range(_GOCH // _GICH):
            sl = pl.ds(j * _GICH, _GICH)
            copies.append(
                pltpu.async_copy(t_hbm.at[dstv.at[sl]], tgv.at[sl], sem_a))
            copies.append(
                pltpu.async_copy(ea_hbm.at[permv.at[sl]], ejv.at[sl], sem_b))
        for c in copies:
            c.wait()
        pltpu.sync_copy(tgv, tg_hbm.at[pl.ds(ob, _GOCH)])
        pltpu.sync_copy(ejv, ej_hbm.at[pl.ds(ob, _GOCH)])
        return carry

    lax.fori_loop(0, _EPW // _GOCH, body, 0)


def _dense_body(ea_ref, tg_ref, ej_ref, wa_ref, bw_ref, ew_ref, cons_ref,
                o_ref):
    ea = ea_ref[...]
    h = tg_ref[...] + jnp.dot(ea, wa_ref[...],
                              preferred_element_type=jnp.float32)
    e2 = jnp.dot(ej_ref[...], bw_ref[...],
                 preferred_element_type=jnp.float32) + cons_ref[0:1, :]
    s = jnp.dot(h + e2, ew_ref[...],
                preferred_element_type=jnp.float32) + cons_ref[1:2, :]
    g = jax.nn.sigmoid(s)
    t = g * h * cons_ref[2:3, :] + cons_ref[3:4, :]
    o_ref[...] = ea + jnp.where(t >= 0, t, 0.01 * t)


def kernel(x, edge_index, edge_attr, edge_mask, A_W, A_b, B_W, B_b, C_W, C_b,
           D_W, D_b, E_W, E_b, bn_gamma, bn_beta, bn_mean, bn_var):
    N, NC = x.shape
    E, EC = edge_attr.shape
    del edge_mask

    dst = edge_index[1]
    src = edge_index[0]

    D1 = D_W[:, :NC]
    D2 = D_W[:, NC:]
    M = C_W.T @ D1.T
    c0 = C_b @ D1.T + A_b @ D2.T + D_b
    WA = A_W.T @ D2.T
    scale = bn_gamma * jax.lax.rsqrt(bn_var + 1e-5)
    shift = bn_beta - bn_mean * scale

    t_tab = pl.pallas_call(
        _node_table_body,
        out_shape=jax.ShapeDtypeStruct((N, EC), jnp.float32),
    )(x, M, c0[None, :])

    mesh = plsc.VectorSubcoreMesh(core_axis_name="c", subcore_axis_name="s")
    sc_params = pltpu.CompilerParams(use_tc_tiling_on_sc=False,
                                     needs_layout_passes=False)

    sort_call = functools.partial(
        pl.kernel,
        out_type=jax.ShapeDtypeStruct((E,), jnp.int32),
        mesh=mesh,
        compiler_params=sc_params,
        scratch_types=[
            pltpu.VMEM_SHARED((_E,), jnp.int32),
            pltpu.VMEM_SHARED((_E,), jnp.int32),
            pltpu.VMEM_SHARED((_E,), jnp.int32),
            pltpu.VMEM_SHARED((_E,), jnp.int32),
            pltpu.VMEM_SHARED((16, _BINS), jnp.int32),
            pltpu.VMEM((_H,), jnp.int32),
            pltpu.VMEM((_H,), jnp.int32),
            pltpu.VMEM((_BINS * 32,), jnp.int32),
            pltpu.VMEM((_BINS * 32,), jnp.int32),
            pltpu.VMEM((16, _BINS), jnp.int32),
            pltpu.VMEM((_BINS,), jnp.int32),
            pltpu.VMEM((_BINS,), jnp.int32),
            pltpu.VMEM((_BINS,), jnp.int32),
            pltpu.VMEM((128,), jnp.int32),
            pltpu.VMEM((128,), jnp.int32),
            pltpu.VMEM((128,), jnp.int32),
            pltpu.VMEM((128,), jnp.int32),
            pltpu.VMEM((128,), jnp.int32),
            pltpu.VMEM((128,), jnp.int32),
            pltpu.VMEM((16,), jnp.int32),
            pltpu.VMEM((16,), jnp.int32),
            pltpu.VMEM((16,), jnp.int32),
            pltpu.SemaphoreType.DMA,
            pltpu.SemaphoreType.DMA,
        ],
    )(_sort_body)
    perm = sort_call(dst, src)

    gather_call = functools.partial(
        pl.kernel,
        out_type=(jax.ShapeDtypeStruct((E, EC), jnp.float32),
                  jax.ShapeDtypeStruct((E, EC), jnp.float32)),
        mesh=mesh,
        compiler_params=sc_params,
        scratch_types=[
            pltpu.VMEM((_GOCH,), jnp.int32),
            pltpu.VMEM((_GOCH,), jnp.int32),
            pltpu.VMEM((_GOCH, EC), jnp.float32),
            pltpu.VMEM((_GOCH, EC), jnp.float32),
            pltpu.SemaphoreType.DMA,
            pltpu.SemaphoreType.DMA,
        ],
    )(_gather_body)
    tg, ej = gather_call(t_tab, edge_attr, dst, perm)

    P = 8 * EC
    R = E // 8
    eye8 = jnp.eye(8, dtype=jnp.float32)
    wa_k = jnp.kron(eye8, WA)
    bw_k = jnp.kron(eye8, B_W.T)
    ew_k = jnp.kron(eye8, E_W.T)
    cons = jnp.stack([
        jnp.tile(B_b, 8), jnp.tile(E_b, 8),
        jnp.tile(scale, 8), jnp.tile(shift, 8)])

    BLK = 2000
    grid = (R // BLK,)
    row_spec = pl.BlockSpec((BLK, P), lambda i: (i, 0))
    full_spec = pl.BlockSpec((P, P), lambda i: (0, 0))
    out = pl.pallas_call(
        _dense_body,
        grid=grid,
        in_specs=[row_spec, row_spec, row_spec, full_spec, full_spec,
                  full_spec, pl.BlockSpec((4, P), lambda i: (0, 0))],
        out_specs=row_spec,
        out_shape=jax.ShapeDtypeStruct((R, P), jnp.float32),
    )(edge_attr.reshape(R, P), tg.reshape(R, P), ej.reshape(R, P),
      wa_k, bw_k, ew_k, cons)

    return out.reshape(E, EC)


# speedup vs baseline: 2.0194x; 1.1135x over previous
import functools

import jax
import jax.numpy as jnp
from jax import lax
from jax.experimental import pallas as pl
from jax.experimental.pallas import tpu as pltpu
from jax.experimental.pallas import tpu_sc as plsc

_E = 320000
_TSL = 20000
_H = 10000
_LBH = 625
_BINS = 256
_SH = (0, 8, 16, 24)
_EPW = 10000
_GOCH = 2000
_GICH = 80


def _node_table_body(x_ref, m_ref, c_ref, o_ref):
    o_ref[...] = (
        jnp.dot(x_ref[...], m_ref[...], preferred_element_type=jnp.float32)
        + c_ref[...]
    )


def _sort_body(dst_hbm, src_hbm, perm_hbm,
               ka_sh, va_sh, kb_sh, vb_sh, ts_sh,
               keych, valch, hist, offs, tsall, totv, pv, gpv,
               kbuf0, vbuf0, pbuf0, kbuf1, vbuf1, pbuf1,
               kbuf2, vbuf2, pbuf2, sema, semb):
    sid = lax.axis_index("s")
    tid = sid
    i16 = lax.iota(jnp.int32, 16)
    lbh = i16 * _LBH
    g0 = tid * _TSL

    def load_half(p, h, src_k, src_v):
        if p == 0:
            pltpu.sync_copy(dst_hbm.at[pl.ds(g0 + h * _H, _H)], keych)
            pltpu.sync_copy(src_hbm.at[pl.ds(g0 + h * _H, _H)], valch)

            def keyinit(v, carry):
                sl = pl.ds(v * 16, 16)
                keych[sl] = keych[sl] * 10000 + valch[sl]
                valch[sl] = g0 + h * _H + v * 16 + i16
                return carry
            lax.fori_loop(0, _H // 16, keyinit, 0)
        else:
            pltpu.sync_copy(src_k.at[pl.ds(g0 + h * _H, _H)], keych)
            pltpu.sync_copy(src_v.at[pl.ds(g0 + h * _H, _H)], valch)

    ones = jnp.ones((16,), jnp.int32)
    for p, sh in enumerate(_SH):
        if p == 0:
            src_k = src_v = None
        elif p % 2 == 1:
            src_k, src_v = ka_sh, va_sh
        else:
            src_k, src_v = kb_sh, vb_sh
        dst_k, dst_v = (ka_sh, va_sh) if p % 2 == 0 else (kb_sh, vb_sh)

        def zero(i, carry):
            hist[pl.ds(i * 16, 16)] = jnp.zeros((16,), jnp.int32)
            return carry
        lax.fori_loop(0, (_BINS * 32) // 16, zero, 0)

        for h in (0, 1):
            load_half(p, h, src_k, src_v)
            grp = h * 16 + i16

            def histo(v, carry):
                kv = plsc.load_gather(keych, [lbh + v])
                digit = (kv >> sh) & (_BINS - 1)
                plsc.addupdate_scatter(hist, [digit * 32 + grp], ones)
                return carry
            lax.fori_loop(0, _LBH, histo, 0)

        def tsum(dc, carry):
            acc = jnp.zeros((16,), jnp.int32)
            dbase = (dc * 16 + i16) * 32
            for g in range(32):
                acc = acc + plsc.load_gather(hist, [dbase + g])
            totv[pl.ds(dc * 16, 16)] = acc
            return carry
        lax.fori_loop(0, _BINS // 16, tsum, 0)
        pltpu.sync_copy(totv, ts_sh.at[tid])
        plsc.subcore_barrier()

        pltpu.sync_copy(ts_sh, tsall)

        def scan1(dc, carry):
            sl = pl.ds(dc * 16, 16)
            tot = jnp.zeros((16,), jnp.int32)
            pfx = jnp.zeros((16,), jnp.int32)
            for t in range(16):
                v = tsall[t, sl]
                tot = tot + v
                pfx = pfx + v * jnp.where(t < tid, 1, 0).astype(jnp.int32)
            totv[sl] = tot
            pv[sl] = pfx
            return carry
        lax.fori_loop(0, _BINS // 16, scan1, 0)

        def scan2(dc, carry):
            sl = pl.ds(dc * 16, 16)
            ch = totv[sl]
            excl = plsc.cumsum(ch) - ch
            gpv[sl] = excl + carry + pv[sl]
            return carry + jnp.sum(ch)
        lax.fori_loop(0, _BINS // 16, scan2, jnp.int32(0))

        def mkoffs(d, carry):
            h0 = plsc.load_gather(hist, [d * 32 + i16])
            h1 = plsc.load_gather(hist, [d * 32 + 16 + i16])
            base = plsc.load_gather(gpv, [jnp.full((16,), d, jnp.int32)])
            offs[pl.ds(d * 32, 16)] = base + (plsc.cumsum(h0) - h0)
            offs[pl.ds(d * 32 + 16, 16)] = (base + jnp.sum(h0)
                                            + (plsc.cumsum(h1) - h1))
            return carry
        lax.fori_loop(0, _BINS, mkoffs, 0)

        for h in (0, 1):
            load_half(p, h, src_k, src_v)
            grp = h * 16 + i16

            def rank1(v):
                idx = lbh + v
                kv = plsc.load_gather(keych, [idx])
                vv = plsc.load_gather(valch, [idx])
                digit = (kv >> sh) & (_BINS - 1)
                b = digit * 32 + grp
                pos = plsc.load_gather(offs, [b])
                plsc.store_scatter(offs, [b], pos + 1)
                return kv, vv, pos

            bufs = ((kbuf0, vbuf0, pbuf0, sema), (kbuf1, vbuf1, pbuf1, semb))

            def rank_chunk(c, kb_, vb_, pb_):
                for u in range(8):
                    kv, vv, pos = rank1(c * 8 + u)
                    usl = pl.ds(u * 16, 16)
                    kb_[usl] = kv
                    vb_[usl] = vv
                    pb_[usl] = pos

            def fire(kb_, vb_, pb_, sem):
                pltpu.async_copy(kb_, dst_k.at[pb_], sem)
                pltpu.async_copy(vb_, dst_v.at[pb_], sem)

            def drain(kb_, vb_, pb_, sem):
                pltpu.make_async_copy(kb_, dst_k.at[pb_], sem).wait()
                pltpu.make_async_copy(vb_, dst_v.at[pb_], sem).wait()

            for s in (0, 1):
                rank_chunk(s, *bufs[s][:3])
                fire(*bufs[s])

            def permute2(c2, carry):
                for s in (0, 1):
                    drain(*bufs[s])
                    rank_chunk(2 + c2 * 2 + s, *bufs[s][:3])
                    fire(*bufs[s])
                return carry
            lax.fori_loop(0, (_LBH // 8 - 2) // 2, permute2, 0)
            for s in (0, 1):
                drain(*bufs[s])

            kv, vv, pos = rank1(_LBH - 1)
            kbuf2[...] = kv
            vbuf2[...] = vv
            pbuf2[...] = pos
            ca = pltpu.async_copy(kbuf2, dst_k.at[pbuf2], sema)
            cb = pltpu.async_copy(vbuf2, dst_v.at[pbuf2], semb)
            ca.wait()
            cb.wait()
        plsc.subcore_barrier()

    pltpu.sync_copy(vb_sh.at[pl.ds(g0, _TSL)],
                    perm_hbm.at[pl.ds(g0, _TSL)])


def _gather_body(t_hbm, ea_hbm, dst_hbm, perm_hbm, tg_hbm, ej_hbm, ea2_hbm,
                 dstv, permv, tgv, ejv, sem_a, sem_b):
    wid = lax.axis_index("s") * 2 + lax.axis_index("c")
    base = wid * _EPW

    def body(o, carry):
        ob = base + o * _GOCH
        pltpu.sync_copy(dst_hbm.at[pl.ds(ob, _GOCH)], dstv)
        pltpu.sync_copy(perm_hbm.at[pl.ds(ob, _GOCH)], permv)
        copies = []
        for j in range(_GOCH // _GICH):
            sl = pl.ds(j * _GICH, _GICH)
            copies.append(
                pltpu.async_copy(t_hbm.at[dstv.at[sl]], tgv.at[sl], sem_a))
            copies.append(
                pltpu.async_copy(ea_hbm.at[permv.at[sl]], ejv.at[sl], sem_b))
        for c in copies:
            c.wait()
        pltpu.sync_copy(tgv, tg_hbm.at[pl.ds(ob, _GOCH)])
        pltpu.sync_copy(ejv, ej_hbm.at[pl.ds(ob, _GOCH)])
        pltpu.sync_copy(ea_hbm.at[pl.ds(ob, _GOCH)], tgv)
        pltpu.sync_copy(tgv, ea2_hbm.at[pl.ds(ob, _GOCH)])
        return carry

    lax.fori_loop(0, _EPW // _GOCH, body, 0)


def _dense_body(ea_ref, tg_ref, ej_ref, wa_ref, bw_ref, ew_ref, cons_ref,
                o_ref):
    ea = ea_ref[...]
    h = tg_ref[...] + jnp.dot(ea, wa_ref[...],
                              preferred_element_type=jnp.float32)
    e2 = jnp.dot(ej_ref[...], bw_ref[...],
                 preferred_element_type=jnp.float32) + cons_ref[0:1, :]
    s = jnp.dot(h + e2, ew_ref[...],
                preferred_element_type=jnp.float32) + cons_ref[1:2, :]
    g = jax.nn.sigmoid(s)
    t = g * h * cons_ref[2:3, :] + cons_ref[3:4, :]
    o_ref[...] = ea + jnp.where(t >= 0, t, 0.01 * t)


def kernel(x, edge_index, edge_attr, edge_mask, A_W, A_b, B_W, B_b, C_W, C_b,
           D_W, D_b, E_W, E_b, bn_gamma, bn_beta, bn_mean, bn_var):
    N, NC = x.shape
    E, EC = edge_attr.shape
    del edge_mask

    dst = edge_index[1]
    src = edge_index[0]

    D1 = D_W[:, :NC]
    D2 = D_W[:, NC:]
    M = C_W.T @ D1.T
    c0 = C_b @ D1.T + A_b @ D2.T + D_b
    WA = A_W.T @ D2.T
    scale = bn_gamma * jax.lax.rsqrt(bn_var + 1e-5)
    shift = bn_beta - bn_mean * scale

    t_tab = pl.pallas_call(
        _node_table_body,
        out_shape=jax.ShapeDtypeStruct((N, EC), jnp.float32),
    )(x, M, c0[None, :])

    mesh = plsc.VectorSubcoreMesh(core_axis_name="c", subcore_axis_name="s")
    sc_params = pltpu.CompilerParams(use_tc_tiling_on_sc=False,
                                     needs_layout_passes=False)

    sort_call = functools.partial(
        pl.kernel,
        out_type=jax.ShapeDtypeStruct((E,), jnp.int32),
        mesh=mesh,
        compiler_params=sc_params,
        scratch_types=[
            pltpu.VMEM_SHARED((_E,), jnp.int32),
            pltpu.VMEM_SHARED((_E,), jnp.int32),
            pltpu.VMEM_SHARED((_E,), jnp.int32),
            pltpu.VMEM_SHARED((_E,), jnp.int32),
            pltpu.VMEM_SHARED((16, _BINS), jnp.int32),
            pltpu.VMEM((_H,), jnp.int32),
            pltpu.VMEM((_H,), jnp.int32),
            pltpu.VMEM((_BINS * 32,), jnp.int32),
            pltpu.VMEM((_BINS * 32,), jnp.int32),
            pltpu.VMEM((16, _BINS), jnp.int32),
            pltpu.VMEM((_BINS,), jnp.int32),
            pltpu.VMEM((_BINS,), jnp.int32),
            pltpu.VMEM((_BINS,), jnp.int32),
            pltpu.VMEM((128,), jnp.int32),
            pltpu.VMEM((128,), jnp.int32),
            pltpu.VMEM((128,), jnp.int32),
            pltpu.VMEM((128,), jnp.int32),
            pltpu.VMEM((128,), jnp.int32),
            pltpu.VMEM((128,), jnp.int32),
            pltpu.VMEM((16,), jnp.int32),
            pltpu.VMEM((16,), jnp.int32),
            pltpu.VMEM((16,), jnp.int32),
            pltpu.SemaphoreType.DMA,
            pltpu.SemaphoreType.DMA,
        ],
    )(_sort_body)
    perm = sort_call(dst, src)

    gather_call = functools.partial(
        pl.kernel,
        out_type=(jax.ShapeDtypeStruct((E, EC), jnp.float32),
                  jax.ShapeDtypeStruct((E, EC), jnp.float32),
                  jax.ShapeDtypeStruct((E, EC), jnp.float32)),
        mesh=mesh,
        compiler_params=sc_params,
        scratch_types=[
            pltpu.VMEM((_GOCH,), jnp.int32),
            pltpu.VMEM((_GOCH,), jnp.int32),
            pltpu.VMEM((_GOCH, EC), jnp.float32),
            pltpu.VMEM((_GOCH, EC), jnp.float32),
            pltpu.SemaphoreType.DMA,
            pltpu.SemaphoreType.DMA,
        ],
    )(_gather_body)
    tg, ej, ea2 = gather_call(t_tab, edge_attr, dst, perm)

    P = 8 * EC
    R = E // 8
    eye8 = jnp.eye(8, dtype=jnp.float32)
    wa_k = jnp.kron(eye8, WA)
    bw_k = jnp.kron(eye8, B_W.T)
    ew_k = jnp.kron(eye8, E_W.T)
    cons = jnp.stack([
        jnp.tile(B_b, 8), jnp.tile(E_b, 8),
        jnp.tile(scale, 8), jnp.tile(shift, 8)])

    BLK = 2000
    grid = (R // BLK,)
    row_spec = pl.BlockSpec((BLK, P), lambda i: (i, 0))
    full_spec = pl.BlockSpec((P, P), lambda i: (0, 0))
    out = pl.pallas_call(
        _dense_body,
        grid=grid,
        in_specs=[row_spec, row_spec, row_spec, full_spec, full_spec,
                  full_spec, pl.BlockSpec((4, P), lambda i: (0, 0))],
        out_specs=row_spec,
        out_shape=jax.ShapeDtypeStruct((R, P), jnp.float32),
    )(ea2.reshape(R, P), tg.reshape(R, P), ej.reshape(R, P),
      wa_k, bw_k, ew_k, cons)

    return out.reshape(E, EC)


# speedup vs baseline: 2.0800x; 1.0300x over previous
import functools

import jax
import jax.numpy as jnp
from jax import lax
from jax.experimental import pallas as pl
from jax.experimental.pallas import tpu as pltpu
from jax.experimental.pallas import tpu_sc as plsc

_E = 320000
_TSL = 20000
_H = 10000
_LBH = 625
_BINS = 256
_SH = (0, 8, 16, 24)
_EPW = 10000
_GOCH = 2000
_GICH = 80


def _node_table_body(x_ref, m_ref, c_ref, o_ref):
    o_ref[...] = (
        jnp.dot(x_ref[...], m_ref[...], preferred_element_type=jnp.float32)
        + c_ref[...]
    )


def _sort_body(dst_hbm, src_hbm, perm_hbm,
               ka_sh, va_sh, kb_sh, vb_sh, ts_sh,
               keych, valch, hist, offs, tsall, totv, pv, gpv,
               kbuf0, vbuf0, pbuf0, kbuf1, vbuf1, pbuf1,
               kbuf2, vbuf2, pbuf2, sema, semb):
    sid = lax.axis_index("s")
    tid = sid
    i16 = lax.iota(jnp.int32, 16)
    lbh = i16 * _LBH
    g0 = tid * _TSL

    def load_half(p, h, src_k, src_v):
        if p == 0:
            c1 = pltpu.async_copy(dst_hbm.at[pl.ds(g0 + h * _H, _H)],
                                  keych, sema)
            c2 = pltpu.async_copy(src_hbm.at[pl.ds(g0 + h * _H, _H)],
                                  valch, semb)
            c1.wait()
            c2.wait()

            def keyinit(v, carry):
                sl = pl.ds(v * 16, 16)
                keych[sl] = keych[sl] * 10000 + valch[sl]
                valch[sl] = g0 + h * _H + v * 16 + i16
                return carry
            lax.fori_loop(0, _H // 16, keyinit, 0)
        else:
            c1 = pltpu.async_copy(src_k.at[pl.ds(g0 + h * _H, _H)],
                                  keych, sema)
            c2 = pltpu.async_copy(src_v.at[pl.ds(g0 + h * _H, _H)],
                                  valch, semb)
            c1.wait()
            c2.wait()

    ones = jnp.ones((16,), jnp.int32)
    for p, sh in enumerate(_SH):
        if p == 0:
            src_k = src_v = None
        elif p % 2 == 1:
            src_k, src_v = ka_sh, va_sh
        else:
            src_k, src_v = kb_sh, vb_sh
        dst_k, dst_v = (ka_sh, va_sh) if p % 2 == 0 else (kb_sh, vb_sh)

        def zero(i, carry):
            hist[pl.ds(i * 16, 16)] = jnp.zeros((16,), jnp.int32)
            return carry
        lax.fori_loop(0, (_BINS * 32) // 16, zero, 0)

        for h in (0, 1):
            load_half(p, h, src_k, src_v)
            grp = h * 16 + i16

            def histo(v, carry):
                kv = plsc.load_gather(keych, [lbh + v])
                digit = (kv >> sh) & (_BINS - 1)
                plsc.addupdate_scatter(hist, [digit * 32 + grp], ones)
                return carry
            lax.fori_loop(0, _LBH, histo, 0)

        def tsum(dc, carry):
            acc = jnp.zeros((16,), jnp.int32)
            dbase = (dc * 16 + i16) * 32
            for g in range(32):
                acc = acc + plsc.load_gather(hist, [dbase + g])
            totv[pl.ds(dc * 16, 16)] = acc
            return carry
        lax.fori_loop(0, _BINS // 16, tsum, 0)
        pltpu.sync_copy(totv, ts_sh.at[tid])
        plsc.subcore_barrier()

        pltpu.sync_copy(ts_sh, tsall)

        def scan1(dc, carry):
            sl = pl.ds(dc * 16, 16)
            tot = jnp.zeros((16,), jnp.int32)
            pfx = jnp.zeros((16,), jnp.int32)
            for t in range(16):
                v = tsall[t, sl]
                tot = tot + v
                pfx = pfx + v * jnp.where(t < tid, 1, 0).astype(jnp.int32)
            totv[sl] = tot
            pv[sl] = pfx
            return carry
        lax.fori_loop(0, _BINS // 16, scan1, 0)

        def scan2(dc, carry):
            sl = pl.ds(dc * 16, 16)
            ch = totv[sl]
            excl = plsc.cumsum(ch) - ch
            gpv[sl] = excl + carry + pv[sl]
            return carry + jnp.sum(ch)
        lax.fori_loop(0, _BINS // 16, scan2, jnp.int32(0))

        def mkoffs(d, carry):
            h0 = plsc.load_gather(hist, [d * 32 + i16])
            h1 = plsc.load_gather(hist, [d * 32 + 16 + i16])
            base = plsc.load_gather(gpv, [jnp.full((16,), d, jnp.int32)])
            offs[pl.ds(d * 32, 16)] = base + (plsc.cumsum(h0) - h0)
            offs[pl.ds(d * 32 + 16, 16)] = (base + jnp.sum(h0)
                                            + (plsc.cumsum(h1) - h1))
            return carry
        lax.fori_loop(0, _BINS, mkoffs, 0)

        for h in (0, 1):
            load_half(p, h, src_k, src_v)
            grp = h * 16 + i16

            def rank1(v):
                idx = lbh + v
                kv = plsc.load_gather(keych, [idx])
                vv = plsc.load_gather(valch, [idx])
                digit = (kv >> sh) & (_BINS - 1)
                b = digit * 32 + grp
                pos = plsc.load_gather(offs, [b])
                plsc.store_scatter(offs, [b], pos + 1)
                return kv, vv, pos

            bufs = ((kbuf0, vbuf0, pbuf0, sema), (kbuf1, vbuf1, pbuf1, semb))

            def rank_chunk(c, kb_, vb_, pb_):
                for u in range(8):
                    kv, vv, pos = rank1(c * 8 + u)
                    usl = pl.ds(u * 16, 16)
                    kb_[usl] = kv
                    vb_[usl] = vv
                    pb_[usl] = pos

            def fire(kb_, vb_, pb_, sem):
                pltpu.async_copy(kb_, dst_k.at[pb_], sem)
                pltpu.async_copy(vb_, dst_v.at[pb_], sem)

            def drain(kb_, vb_, pb_, sem):
                pltpu.make_async_copy(kb_, dst_k.at[pb_], sem).wait()
                pltpu.make_async_copy(vb_, dst_v.at[pb_], sem).wait()

            for s in (0, 1):
                rank_chunk(s, *bufs[s][:3])
                fire(*bufs[s])

            def permute2(c2, carry):
                for s in (0, 1):
                    drain(*bufs[s])
                    rank_chunk(2 + c2 * 2 + s, *bufs[s][:3])
                    fire(*bufs[s])
                return carry
            lax.fori_loop(0, (_LBH // 8 - 2) // 2, permute2, 0)
            for s in (0, 1):
                drain(*bufs[s])

            kv, vv, pos = rank1(_LBH - 1)
            kbuf2[...] = kv
            vbuf2[...] = vv
            pbuf2[...] = pos
            ca = pltpu.async_copy(kbuf2, dst_k.at[pbuf2], sema)
            cb = pltpu.async_copy(vbuf2, dst_v.at[pbuf2], semb)
            ca.wait()
            cb.wait()
        plsc.subcore_barrier()

    pltpu.sync_copy(vb_sh.at[pl.ds(g0, _TSL)],
                    perm_hbm.at[pl.ds(g0, _TSL)])


def _gather_body(t_hbm, ea_hbm, dst_hbm, perm_hbm, tg_hbm, ej_hbm, ea2_hbm,
                 dstv, permv, tgv, ejv, eav, sem_a, sem_b, sem_c):
    wid = lax.axis_index("s") * 2 + lax.axis_index("c")
    base = wid * _EPW

    def body(o, carry):
        ob = base + o * _GOCH
        pltpu.sync_copy(dst_hbm.at[pl.ds(ob, _GOCH)], dstv)
        pltpu.sync_copy(perm_hbm.at[pl.ds(ob, _GOCH)], permv)
        copies = []
        for j in range(_GOCH // _GICH):
            sl = pl.ds(j * _GICH, _GICH)
            copies.append(
                pltpu.async_copy(t_hbm.at[dstv.at[sl]], tgv.at[sl], sem_a))
            copies.append(
                pltpu.async_copy(ea_hbm.at[permv.at[sl]], ejv.at[sl], sem_b))
        cea = pltpu.async_copy(ea_hbm.at[pl.ds(ob, _GOCH)], eav, sem_c)
        for c in copies:
            c.wait()
        cea.wait()
        pltpu.sync_copy(tgv, tg_hbm.at[pl.ds(ob, _GOCH)])
        pltpu.sync_copy(ejv, ej_hbm.at[pl.ds(ob, _GOCH)])
        pltpu.sync_copy(eav, ea2_hbm.at[pl.ds(ob, _GOCH)])
        return carry

    lax.fori_loop(0, _EPW // _GOCH, body, 0)


def _dense_body(ea_ref, tg_ref, ej_ref, wa_ref, bw_ref, ew_ref, cons_ref,
                o_ref):
    ea = ea_ref[...]
    h = tg_ref[...] + jnp.dot(ea, wa_ref[...],
                              preferred_element_type=jnp.float32)
    e2 = jnp.dot(ej_ref[...], bw_ref[...],
                 preferred_element_type=jnp.float32) + cons_ref[0:1, :]
    s = jnp.dot(h + e2, ew_ref[...],
                preferred_element_type=jnp.float32) + cons_ref[1:2, :]
    g = jax.nn.sigmoid(s)
    t = g * h * cons_ref[2:3, :] + cons_ref[3:4, :]
    o_ref[...] = ea + jnp.where(t >= 0, t, 0.01 * t)


def kernel(x, edge_index, edge_attr, edge_mask, A_W, A_b, B_W, B_b, C_W, C_b,
           D_W, D_b, E_W, E_b, bn_gamma, bn_beta, bn_mean, bn_var):
    N, NC = x.shape
    E, EC = edge_attr.shape
    del edge_mask

    dst = edge_index[1]
    src = edge_index[0]

    D1 = D_W[:, :NC]
    D2 = D_W[:, NC:]
    M = C_W.T @ D1.T
    c0 = C_b @ D1.T + A_b @ D2.T + D_b
    WA = A_W.T @ D2.T
    scale = bn_gamma * jax.lax.rsqrt(bn_var + 1e-5)
    shift = bn_beta - bn_mean * scale

    t_tab = pl.pallas_call(
        _node_table_body,
        out_shape=jax.ShapeDtypeStruct((N, EC), jnp.float32),
    )(x, M, c0[None, :])

    mesh = plsc.VectorSubcoreMesh(core_axis_name="c", subcore_axis_name="s")
    sc_params = pltpu.CompilerParams(use_tc_tiling_on_sc=False,
                                     needs_layout_passes=False)

    sort_call = functools.partial(
        pl.kernel,
        out_type=jax.ShapeDtypeStruct((E,), jnp.int32),
        mesh=mesh,
        compiler_params=sc_params,
        scratch_types=[
            pltpu.VMEM_SHARED((_E,), jnp.int32),
            pltpu.VMEM_SHARED((_E,), jnp.int32),
            pltpu.VMEM_SHARED((_E,), jnp.int32),
            pltpu.VMEM_SHARED((_E,), jnp.int32),
            pltpu.VMEM_SHARED((16, _BINS), jnp.int32),
            pltpu.VMEM((_H,), jnp.int32),
            pltpu.VMEM((_H,), jnp.int32),
            pltpu.VMEM((_BINS * 32,), jnp.int32),
            pltpu.VMEM((_BINS * 32,), jnp.int32),
            pltpu.VMEM((16, _BINS), jnp.int32),
            pltpu.VMEM((_BINS,), jnp.int32),
            pltpu.VMEM((_BINS,), jnp.int32),
            pltpu.VMEM((_BINS,), jnp.int32),
            pltpu.VMEM((128,), jnp.int32),
            pltpu.VMEM((128,), jnp.int32),
            pltpu.VMEM((128,), jnp.int32),
            pltpu.VMEM((128,), jnp.int32),
            pltpu.VMEM((128,), jnp.int32),
            pltpu.VMEM((128,), jnp.int32),
            pltpu.VMEM((16,), jnp.int32),
            pltpu.VMEM((16,), jnp.int32),
            pltpu.VMEM((16,), jnp.int32),
            pltpu.SemaphoreType.DMA,
            pltpu.SemaphoreType.DMA,
        ],
    )(_sort_body)
    perm = sort_call(dst, src)

    gather_call = functools.partial(
        pl.kernel,
        out_type=(jax.ShapeDtypeStruct((E, EC), jnp.float32),
                  jax.ShapeDtypeStruct((E, EC), jnp.float32),
                  jax.ShapeDtypeStruct((E, EC), jnp.float32)),
        mesh=mesh,
        compiler_params=sc_params,
        scratch_types=[
            pltpu.VMEM((_GOCH,), jnp.int32),
            pltpu.VMEM((_GOCH,), jnp.int32),
            pltpu.VMEM((_GOCH, EC), jnp.float32),
            pltpu.VMEM((_GOCH, EC), jnp.float32),
            pltpu.VMEM((_GOCH, EC), jnp.float32),
            pltpu.SemaphoreType.DMA,
            pltpu.SemaphoreType.DMA,
            pltpu.SemaphoreType.DMA,
        ],
    )(_gather_body)
    tg, ej, ea2 = gather_call(t_tab, edge_attr, dst, perm)

    P = 8 * EC
    R = E // 8
    eye8 = jnp.eye(8, dtype=jnp.float32)
    wa_k = jnp.kron(eye8, WA)
    bw_k = jnp.kron(eye8, B_W.T)
    ew_k = jnp.kron(eye8, E_W.T)
    cons = jnp.stack([
        jnp.tile(B_b, 8), jnp.tile(E_b, 8),
        jnp.tile(scale, 8), jnp.tile(shift, 8)])

    BLK = 4000
    grid = (R // BLK,)
    row_spec = pl.BlockSpec((BLK, P), lambda i: (i, 0))
    full_spec = pl.BlockSpec((P, P), lambda i: (0, 0))
    out = pl.pallas_call(
        _dense_body,
        grid=grid,
        in_specs=[row_spec, row_spec, row_spec, full_spec, full_spec,
                  full_spec, pl.BlockSpec((4, P), lambda i: (0, 0))],
        out_specs=row_spec,
        out_shape=jax.ShapeDtypeStruct((R, P), jnp.float32),
    )(ea2.reshape(R, P), tg.reshape(R, P), ej.reshape(R, P),
      wa_k, bw_k, ew_k, cons)

    return out.reshape(E, EC)


# speedup vs baseline: 2.1084x; 1.0137x over previous
import functools

import jax
import jax.numpy as jnp
from jax import lax
from jax.experimental import pallas as pl
from jax.experimental.pallas import tpu as pltpu
from jax.experimental.pallas import tpu_sc as plsc

_E = 320000
_TSL = 20000
_H = 10000
_LBH = 625
_BINS = 256
_SH = (0, 8, 16, 24)
_EPW = 10000
_GOCH = 2000
_GICH = 80


def _node_table_body(x_ref, m_ref, c_ref, o_ref):
    o_ref[...] = (
        jnp.dot(x_ref[...], m_ref[...], preferred_element_type=jnp.float32)
        + c_ref[...]
    )


def _sort_body(dst_hbm, src_hbm, perm_hbm,
               ka_sh, va_sh, kb_sh, vb_sh, ts_sh,
               keych, valch, hist, offs, tsall, totv, pv, gpv,
               kbuf0, vbuf0, pbuf0, kbuf1, vbuf1, pbuf1,
               kbuf2, vbuf2, pbuf2, sema, semb):
    sid = lax.axis_index("s")
    tid = sid
    i16 = lax.iota(jnp.int32, 16)
    lbh = i16 * _LBH
    g0 = tid * _TSL

    def load_half(p, h, src_k, src_v):
        if p == 0:
            c1 = pltpu.async_copy(dst_hbm.at[pl.ds(g0 + h * _H, _H)],
                                  keych, sema)
            c2 = pltpu.async_copy(src_hbm.at[pl.ds(g0 + h * _H, _H)],
                                  valch, semb)
            c1.wait()
            c2.wait()

            def keyinit(v, carry):
                for u in range(5):
                    sl = pl.ds((v * 5 + u) * 16, 16)
                    keych[sl] = keych[sl] * 10000 + valch[sl]
                    valch[sl] = g0 + h * _H + (v * 5 + u) * 16 + i16
                return carry
            lax.fori_loop(0, _H // 80, keyinit, 0)
        else:
            c1 = pltpu.async_copy(src_k.at[pl.ds(g0 + h * _H, _H)],
                                  keych, sema)
            c2 = pltpu.async_copy(src_v.at[pl.ds(g0 + h * _H, _H)],
                                  valch, semb)
            c1.wait()
            c2.wait()

    ones = jnp.ones((16,), jnp.int32)
    for p, sh in enumerate(_SH):
        if p == 0:
            src_k = src_v = None
        elif p % 2 == 1:
            src_k, src_v = ka_sh, va_sh
        else:
            src_k, src_v = kb_sh, vb_sh
        dst_k, dst_v = (ka_sh, va_sh) if p % 2 == 0 else (kb_sh, vb_sh)

        def zero(i, carry):
            hist[pl.ds(i * 16, 16)] = jnp.zeros((16,), jnp.int32)
            return carry
        lax.fori_loop(0, (_BINS * 32) // 16, zero, 0)

        for h in (0, 1):
            load_half(p, h, src_k, src_v)
            grp = h * 16 + i16

            def histo(v, carry):
                for u in range(5):
                    kv = plsc.load_gather(keych, [lbh + v * 5 + u])
                    digit = (kv >> sh) & (_BINS - 1)
                    plsc.addupdate_scatter(hist, [digit * 32 + grp], ones)
                return carry
            lax.fori_loop(0, _LBH // 5, histo, 0)

        def tsum(dc, carry):
            acc = jnp.zeros((16,), jnp.int32)
            dbase = (dc * 16 + i16) * 32
            for g in range(32):
                acc = acc + plsc.load_gather(hist, [dbase + g])
            totv[pl.ds(dc * 16, 16)] = acc
            return carry
        lax.fori_loop(0, _BINS // 16, tsum, 0)
        pltpu.sync_copy(totv, ts_sh.at[tid])
        plsc.subcore_barrier()

        pltpu.sync_copy(ts_sh, tsall)

        def scan1(dc, carry):
            sl = pl.ds(dc * 16, 16)
            tot = jnp.zeros((16,), jnp.int32)
            pfx = jnp.zeros((16,), jnp.int32)
            for t in range(16):
                v = tsall[t, sl]
                tot = tot + v
                pfx = pfx + v * jnp.where(t < tid, 1, 0).astype(jnp.int32)
            totv[sl] = tot
            pv[sl] = pfx
            return carry
        lax.fori_loop(0, _BINS // 16, scan1, 0)

        def scan2(dc, carry):
            sl = pl.ds(dc * 16, 16)
            ch = totv[sl]
            excl = plsc.cumsum(ch) - ch
            gpv[sl] = excl + carry + pv[sl]
            return carry + jnp.sum(ch)
        lax.fori_loop(0, _BINS // 16, scan2, jnp.int32(0))

        def mkoffs(d, carry):
            h0 = plsc.load_gather(hist, [d * 32 + i16])
            h1 = plsc.load_gather(hist, [d * 32 + 16 + i16])
            base = plsc.load_gather(gpv, [jnp.full((16,), d, jnp.int32)])
            offs[pl.ds(d * 32, 16)] = base + (plsc.cumsum(h0) - h0)
            offs[pl.ds(d * 32 + 16, 16)] = (base + jnp.sum(h0)
                                            + (plsc.cumsum(h1) - h1))
            return carry
        lax.fori_loop(0, _BINS, mkoffs, 0)

        for h in (0, 1):
            load_half(p, h, src_k, src_v)
            grp = h * 16 + i16

            def rank1(v):
                idx = lbh + v
                kv = plsc.load_gather(keych, [idx])
                vv = plsc.load_gather(valch, [idx])
                digit = (kv >> sh) & (_BINS - 1)
                b = digit * 32 + grp
                pos = plsc.load_gather(offs, [b])
                plsc.store_scatter(offs, [b], pos + 1)
                return kv, vv, pos

            bufs = ((kbuf0, vbuf0, pbuf0, sema), (kbuf1, vbuf1, pbuf1, semb))

            def rank_chunk(c, kb_, vb_, pb_):
                for u in range(8):
                    kv, vv, pos = rank1(c * 8 + u)
                    usl = pl.ds(u * 16, 16)
                    kb_[usl] = kv
                    vb_[usl] = vv
                    pb_[usl] = pos

            def fire(kb_, vb_, pb_, sem):
                pltpu.async_copy(kb_, dst_k.at[pb_], sem)
                pltpu.async_copy(vb_, dst_v.at[pb_], sem)

            def drain(kb_, vb_, pb_, sem):
                pltpu.make_async_copy(kb_, dst_k.at[pb_], sem).wait()
                pltpu.make_async_copy(vb_, dst_v.at[pb_], sem).wait()

            for s in (0, 1):
                rank_chunk(s, *bufs[s][:3])
                fire(*bufs[s])

            def permute2(c2, carry):
                for s in (0, 1):
                    drain(*bufs[s])
                    rank_chunk(2 + c2 * 2 + s, *bufs[s][:3])
                    fire(*bufs[s])
                return carry
            lax.fori_loop(0, (_LBH // 8 - 2) // 2, permute2, 0)
            for s in (0, 1):
                drain(*bufs[s])

            kv, vv, pos = rank1(_LBH - 1)
            kbuf2[...] = kv
            vbuf2[...] = vv
            pbuf2[...] = pos
            ca = pltpu.async_copy(kbuf2, dst_k.at[pbuf2], sema)
            cb = pltpu.async_copy(vbuf2, dst_v.at[pbuf2], semb)
            ca.wait()
            cb.wait()
        plsc.subcore_barrier()

    pltpu.sync_copy(vb_sh.at[pl.ds(g0, _TSL)],
                    perm_hbm.at[pl.ds(g0, _TSL)])


def _gather_body(t_hbm, ea_hbm, dst_hbm, perm_hbm, tg_hbm, ej_hbm, ea2_hbm,
                 dstv, permv, tgv, ejv, eav, sem_a, sem_b, sem_c):
    wid = lax.axis_index("s") * 2 + lax.axis_index("c")
    base = wid * _EPW

    def body(o, carry):
        ob = base + o * _GOCH
        pltpu.sync_copy(dst_hbm.at[pl.ds(ob, _GOCH)], dstv)
        pltpu.sync_copy(perm_hbm.at[pl.ds(ob, _GOCH)], permv)
        copies = []
        for j in range(_GOCH // _GICH):
            sl = pl.ds(j * _GICH, _GICH)
            copies.append(
                pltpu.async_copy(t_hbm.at[dstv.at[sl]], tgv.at[sl], sem_a))
            copies.append(
                pltpu.async_copy(ea_hbm.at[permv.at[sl]], ejv.at[sl], sem_b))
        cea = pltpu.async_copy(ea_hbm.at[pl.ds(ob, _GOCH)], eav, sem_c)
        for c in copies:
            c.wait()
        cea.wait()
        pltpu.sync_copy(tgv, tg_hbm.at[pl.ds(ob, _GOCH)])
        pltpu.sync_copy(ejv, ej_hbm.at[pl.ds(ob, _GOCH)])
        pltpu.sync_copy(eav, ea2_hbm.at[pl.ds(ob, _GOCH)])
        return carry

    lax.fori_loop(0, _EPW // _GOCH, body, 0)


def _dense_body(ea_ref, tg_ref, ej_ref, wa_ref, bw_ref, ew_ref, cons_ref,
                o_ref):
    ea = ea_ref[...]
    h = tg_ref[...] + jnp.dot(ea, wa_ref[...],
                              preferred_element_type=jnp.float32)
    e2 = jnp.dot(ej_ref[...], bw_ref[...],
                 preferred_element_type=jnp.float32) + cons_ref[0:1, :]
    s = jnp.dot(h + e2, ew_ref[...],
                preferred_element_type=jnp.float32) + cons_ref[1:2, :]
    g = jax.nn.sigmoid(s)
    t = g * h * cons_ref[2:3, :] + cons_ref[3:4, :]
    o_ref[...] = ea + jnp.where(t >= 0, t, 0.01 * t)


def kernel(x, edge_index, edge_attr, edge_mask, A_W, A_b, B_W, B_b, C_W, C_b,
           D_W, D_b, E_W, E_b, bn_gamma, bn_beta, bn_mean, bn_var):
    N, NC = x.shape
    E, EC = edge_attr.shape
    del edge_mask

    dst = edge_index[1]
    src = edge_index[0]

    D1 = D_W[:, :NC]
    D2 = D_W[:, NC:]
    M = C_W.T @ D1.T
    c0 = C_b @ D1.T + A_b @ D2.T + D_b
    WA = A_W.T @ D2.T
    scale = bn_gamma * jax.lax.rsqrt(bn_var + 1e-5)
    shift = bn_beta - bn_mean * scale

    t_tab = pl.pallas_call(
        _node_table_body,
        out_shape=jax.ShapeDtypeStruct((N, EC), jnp.float32),
    )(x, M, c0[None, :])

    mesh = plsc.VectorSubcoreMesh(core_axis_name="c", subcore_axis_name="s")
    sc_params = pltpu.CompilerParams(use_tc_tiling_on_sc=False,
                                     needs_layout_passes=False)

    sort_call = functools.partial(
        pl.kernel,
        out_type=jax.ShapeDtypeStruct((E,), jnp.int32),
        mesh=mesh,
        compiler_params=sc_params,
        scratch_types=[
            pltpu.VMEM_SHARED((_E,), jnp.int32),
            pltpu.VMEM_SHARED((_E,), jnp.int32),
            pltpu.VMEM_SHARED((_E,), jnp.int32),
            pltpu.VMEM_SHARED((_E,), jnp.int32),
            pltpu.VMEM_SHARED((16, _BINS), jnp.int32),
            pltpu.VMEM((_H,), jnp.int32),
            pltpu.VMEM((_H,), jnp.int32),
            pltpu.VMEM((_BINS * 32,), jnp.int32),
            pltpu.VMEM((_BINS * 32,), jnp.int32),
            pltpu.VMEM((16, _BINS), jnp.int32),
            pltpu.VMEM((_BINS,), jnp.int32),
            pltpu.VMEM((_BINS,), jnp.int32),
            pltpu.VMEM((_BINS,), jnp.int32),
            pltpu.VMEM((128,), jnp.int32),
            pltpu.VMEM((128,), jnp.int32),
            pltpu.VMEM((128,), jnp.int32),
            pltpu.VMEM((128,), jnp.int32),
            pltpu.VMEM((128,), jnp.int32),
            pltpu.VMEM((128,), jnp.int32),
            pltpu.VMEM((16,), jnp.int32),
            pltpu.VMEM((16,), jnp.int32),
            pltpu.VMEM((16,), jnp.int32),
            pltpu.SemaphoreType.DMA,
            pltpu.SemaphoreType.DMA,
        ],
    )(_sort_body)
    perm = sort_call(dst, src)

    gather_call = functools.partial(
        pl.kernel,
        out_type=(jax.ShapeDtypeStruct((E, EC), jnp.float32),
                  jax.ShapeDtypeStruct((E, EC), jnp.float32),
                  jax.ShapeDtypeStruct((E, EC), jnp.float32)),
        mesh=mesh,
        compiler_params=sc_params,
        scratch_types=[
            pltpu.VMEM((_GOCH,), jnp.int32),
            pltpu.VMEM((_GOCH,), jnp.int32),
            pltpu.VMEM((_GOCH, EC), jnp.float32),
            pltpu.VMEM((_GOCH, EC), jnp.float32),
            pltpu.VMEM((_GOCH, EC), jnp.float32),
            pltpu.SemaphoreType.DMA,
            pltpu.SemaphoreType.DMA,
            pltpu.SemaphoreType.DMA,
        ],
    )(_gather_body)
    tg, ej, ea2 = gather_call(t_tab, edge_attr, dst, perm)

    P = 8 * EC
    R = E // 8
    eye8 = jnp.eye(8, dtype=jnp.float32)
    wa_k = jnp.kron(eye8, WA)
    bw_k = jnp.kron(eye8, B_W.T)
    ew_k = jnp.kron(eye8, E_W.T)
    cons = jnp.stack([
        jnp.tile(B_b, 8), jnp.tile(E_b, 8),
        jnp.tile(scale, 8), jnp.tile(shift, 8)])

    BLK = 4000
    grid = (R // BLK,)
    row_spec = pl.BlockSpec((BLK, P), lambda i: (i, 0))
    full_spec = pl.BlockSpec((P, P), lambda i: (0, 0))
    out = pl.pallas_call(
        _dense_body,
        grid=grid,
        in_specs=[row_spec, row_spec, row_spec, full_spec, full_spec,
                  full_spec, pl.BlockSpec((4, P), lambda i: (0, 0))],
        out_specs=row_spec,
        out_shape=jax.ShapeDtypeStruct((R, P), jnp.float32),
    )(ea2.reshape(R, P), tg.reshape(R, P), ej.reshape(R, P),
      wa_k, bw_k, ew_k, cons)

    return out.reshape(E, EC)


# speedup vs baseline: 2.1381x; 1.0140x over previous
import functools

import jax
import jax.numpy as jnp
from jax import lax
from jax.experimental import pallas as pl
from jax.experimental.pallas import tpu as pltpu
from jax.experimental.pallas import tpu_sc as plsc

_E = 320000
_TSL = 20000
_H = 10000
_LBH = 625
_BINS = 256
_SH = (0, 8, 16, 24)
_EPW = 10000
_GOCH = 2000
_GICH = 80


def _node_table_body(x_ref, m_ref, c_ref, o_ref):
    o_ref[...] = (
        jnp.dot(x_ref[...], m_ref[...], preferred_element_type=jnp.float32)
        + c_ref[...]
    )


def _sort_body(dst_hbm, src_hbm, perm_hbm,
               ka_sh, va_sh, kb_sh, vb_sh, ts_sh,
               keych, valch, hist, offs, tsall, totv, pv, gpv,
               kbuf0, vbuf0, pbuf0, kbuf1, vbuf1, pbuf1,
               kbuf2, vbuf2, pbuf2, sema, semb):
    sid = lax.axis_index("s")
    tid = sid
    i16 = lax.iota(jnp.int32, 16)
    lbh = i16 * _LBH
    g0 = tid * _TSL

    def load_half(p, h, src_k, src_v):
        if p == 0:
            c1 = pltpu.async_copy(dst_hbm.at[pl.ds(g0 + h * _H, _H)],
                                  keych, sema)
            c2 = pltpu.async_copy(src_hbm.at[pl.ds(g0 + h * _H, _H)],
                                  valch, semb)
            c1.wait()
            c2.wait()

            def keyinit(v, carry):
                for u in range(5):
                    sl = pl.ds((v * 5 + u) * 16, 16)
                    keych[sl] = keych[sl] * 10000 + valch[sl]
                    valch[sl] = g0 + h * _H + (v * 5 + u) * 16 + i16
                return carry
            lax.fori_loop(0, _H // 80, keyinit, 0)
        else:
            c1 = pltpu.async_copy(src_k.at[pl.ds(g0 + h * _H, _H)],
                                  keych, sema)
            c2 = pltpu.async_copy(src_v.at[pl.ds(g0 + h * _H, _H)],
                                  valch, semb)
            c1.wait()
            c2.wait()

    ones = jnp.ones((16,), jnp.int32)
    for p, sh in enumerate(_SH):
        if p == 0:
            src_k = src_v = None
        elif p % 2 == 1:
            src_k, src_v = ka_sh, va_sh
        else:
            src_k, src_v = kb_sh, vb_sh
        dst_k, dst_v = (ka_sh, va_sh) if p % 2 == 0 else (kb_sh, vb_sh)

        def zero(i, carry):
            for u in range(8):
                hist[pl.ds((i * 8 + u) * 16, 16)] = jnp.zeros((16,),
                                                              jnp.int32)
            return carry
        lax.fori_loop(0, (_BINS * 32) // 128, zero, 0)

        for h in (0, 1):
            load_half(p, h, src_k, src_v)
            grp = h * 16 + i16

            def histo(v, carry):
                for u in range(5):
                    kv = plsc.load_gather(keych, [lbh + v * 5 + u])
                    digit = (kv >> sh) & (_BINS - 1)
                    plsc.addupdate_scatter(hist, [digit * 32 + grp], ones)
                return carry
            lax.fori_loop(0, _LBH // 5, histo, 0)

        def tsum(dc, carry):
            acc = jnp.zeros((16,), jnp.int32)
            dbase = (dc * 16 + i16) * 32
            for g in range(32):
                acc = acc + plsc.load_gather(hist, [dbase + g])
            totv[pl.ds(dc * 16, 16)] = acc
            return carry
        lax.fori_loop(0, _BINS // 16, tsum, 0)
        pltpu.sync_copy(totv, ts_sh.at[tid])
        plsc.subcore_barrier()

        pltpu.sync_copy(ts_sh, tsall)

        def scan1(dc, carry):
            sl = pl.ds(dc * 16, 16)
            tot = jnp.zeros((16,), jnp.int32)
            pfx = jnp.zeros((16,), jnp.int32)
            for t in range(16):
                v = tsall[t, sl]
                tot = tot + v
                pfx = pfx + v * jnp.where(t < tid, 1, 0).astype(jnp.int32)
            totv[sl] = tot
            pv[sl] = pfx
            return carry
        lax.fori_loop(0, _BINS // 16, scan1, 0)

        def scan2(dc, carry):
            sl = pl.ds(dc * 16, 16)
            ch = totv[sl]
            excl = plsc.cumsum(ch) - ch
            gpv[sl] = excl + carry + pv[sl]
            return carry + jnp.sum(ch)
        lax.fori_loop(0, _BINS // 16, scan2, jnp.int32(0))

        def mkoffs(d4, carry):
            for u in range(4):
                d = d4 * 4 + u
                h0 = plsc.load_gather(hist, [d * 32 + i16])
                h1 = plsc.load_gather(hist, [d * 32 + 16 + i16])
                base = plsc.load_gather(gpv,
                                        [jnp.full((16,), d, jnp.int32)])
                offs[pl.ds(d * 32, 16)] = base + (plsc.cumsum(h0) - h0)
                offs[pl.ds(d * 32 + 16, 16)] = (base + jnp.sum(h0)
                                                + (plsc.cumsum(h1) - h1))
            return carry
        lax.fori_loop(0, _BINS // 4, mkoffs, 0)

        for h in (0, 1):
            load_half(p, h, src_k, src_v)
            grp = h * 16 + i16

            def rank1(v):
                idx = lbh + v
                kv = plsc.load_gather(keych, [idx])
                vv = plsc.load_gather(valch, [idx])
                digit = (kv >> sh) & (_BINS - 1)
                b = digit * 32 + grp
                pos = plsc.load_gather(offs, [b])
                plsc.store_scatter(offs, [b], pos + 1)
                return kv, vv, pos

            bufs = ((kbuf0, vbuf0, pbuf0, sema), (kbuf1, vbuf1, pbuf1, semb))

            def rank_chunk(c, kb_, vb_, pb_):
                for u in range(8):
                    kv, vv, pos = rank1(c * 8 + u)
                    usl = pl.ds(u * 16, 16)
                    kb_[usl] = kv
                    vb_[usl] = vv
                    pb_[usl] = pos

            def fire(kb_, vb_, pb_, sem):
                pltpu.async_copy(kb_, dst_k.at[pb_], sem)
                pltpu.async_copy(vb_, dst_v.at[pb_], sem)

            def drain(kb_, vb_, pb_, sem):
                pltpu.make_async_copy(kb_, dst_k.at[pb_], sem).wait()
                pltpu.make_async_copy(vb_, dst_v.at[pb_], sem).wait()

            for s in (0, 1):
                rank_chunk(s, *bufs[s][:3])
                fire(*bufs[s])

            def permute2(c2, carry):
                for s in (0, 1):
                    drain(*bufs[s])
                    rank_chunk(2 + c2 * 2 + s, *bufs[s][:3])
                    fire(*bufs[s])
                return carry
            lax.fori_loop(0, (_LBH // 8 - 2) // 2, permute2, 0)
            for s in (0, 1):
                drain(*bufs[s])

            kv, vv, pos = rank1(_LBH - 1)
            kbuf2[...] = kv
            vbuf2[...] = vv
            pbuf2[...] = pos
            ca = pltpu.async_copy(kbuf2, dst_k.at[pbuf2], sema)
            cb = pltpu.async_copy(vbuf2, dst_v.at[pbuf2], semb)
            ca.wait()
            cb.wait()
        plsc.subcore_barrier()

    pltpu.sync_copy(vb_sh.at[pl.ds(g0, _TSL)],
                    perm_hbm.at[pl.ds(g0, _TSL)])


def _gather_body(t_hbm, ea_hbm, dst_hbm, perm_hbm, tg_hbm, ej_hbm, ea2_hbm,
                 dstv, permv, tgv, ejv, eav, sem_a, sem_b, sem_c):
    wid = lax.axis_index("s") * 2 + lax.axis_index("c")
    base = wid * _EPW

    def body(o, carry):
        ob = base + o * _GOCH
        pltpu.sync_copy(dst_hbm.at[pl.ds(ob, _GOCH)], dstv)
        pltpu.sync_copy(perm_hbm.at[pl.ds(ob, _GOCH)], permv)
        copies = []
        for j in range(_GOCH // _GICH):
            sl = pl.ds(j * _GICH, _GICH)
            copies.append(
                pltpu.async_copy(t_hbm.at[dstv.at[sl]], tgv.at[sl], sem_a))
            copies.append(
                pltpu.async_copy(ea_hbm.at[permv.at[sl]], ejv.at[sl], sem_b))
        cea = pltpu.async_copy(ea_hbm.at[pl.ds(ob, _GOCH)], eav, sem_c)
        for c in copies:
            c.wait()
        cea.wait()
        pltpu.sync_copy(tgv, tg_hbm.at[pl.ds(ob, _GOCH)])
        pltpu.sync_copy(ejv, ej_hbm.at[pl.ds(ob, _GOCH)])
        pltpu.sync_copy(eav, ea2_hbm.at[pl.ds(ob, _GOCH)])
        return carry

    lax.fori_loop(0, _EPW // _GOCH, body, 0)


def _dense_body(ea_ref, tg_ref, ej_ref, wa_ref, bw_ref, ew_ref, cons_ref,
                o_ref):
    ea = ea_ref[...]
    h = tg_ref[...] + jnp.dot(ea, wa_ref[...],
                              preferred_element_type=jnp.float32)
    e2 = jnp.dot(ej_ref[...], bw_ref[...],
                 preferred_element_type=jnp.float32) + cons_ref[0:1, :]
    s = jnp.dot(h + e2, ew_ref[...],
                preferred_element_type=jnp.float32) + cons_ref[1:2, :]
    g = jax.nn.sigmoid(s)
    t = g * h * cons_ref[2:3, :] + cons_ref[3:4, :]
    o_ref[...] = ea + jnp.where(t >= 0, t, 0.01 * t)


def kernel(x, edge_index, edge_attr, edge_mask, A_W, A_b, B_W, B_b, C_W, C_b,
           D_W, D_b, E_W, E_b, bn_gamma, bn_beta, bn_mean, bn_var):
    N, NC = x.shape
    E, EC = edge_attr.shape
    del edge_mask

    dst = edge_index[1]
    src = edge_index[0]

    D1 = D_W[:, :NC]
    D2 = D_W[:, NC:]
    M = C_W.T @ D1.T
    c0 = C_b @ D1.T + A_b @ D2.T + D_b
    WA = A_W.T @ D2.T
    scale = bn_gamma * jax.lax.rsqrt(bn_var + 1e-5)
    shift = bn_beta - bn_mean * scale

    t_tab = pl.pallas_call(
        _node_table_body,
        out_shape=jax.ShapeDtypeStruct((N, EC), jnp.float32),
    )(x, M, c0[None, :])

    mesh = plsc.VectorSubcoreMesh(core_axis_name="c", subcore_axis_name="s")
    sc_params = pltpu.CompilerParams(use_tc_tiling_on_sc=False,
                                     needs_layout_passes=False)

    sort_call = functools.partial(
        pl.kernel,
        out_type=jax.ShapeDtypeStruct((E,), jnp.int32),
        mesh=mesh,
        compiler_params=sc_params,
        scratch_types=[
            pltpu.VMEM_SHARED((_E,), jnp.int32),
            pltpu.VMEM_SHARED((_E,), jnp.int32),
            pltpu.VMEM_SHARED((_E,), jnp.int32),
            pltpu.VMEM_SHARED((_E,), jnp.int32),
            pltpu.VMEM_SHARED((16, _BINS), jnp.int32),
            pltpu.VMEM((_H,), jnp.int32),
            pltpu.VMEM((_H,), jnp.int32),
            pltpu.VMEM((_BINS * 32,), jnp.int32),
            pltpu.VMEM((_BINS * 32,), jnp.int32),
            pltpu.VMEM((16, _BINS), jnp.int32),
            pltpu.VMEM((_BINS,), jnp.int32),
            pltpu.VMEM((_BINS,), jnp.int32),
            pltpu.VMEM((_BINS,), jnp.int32),
            pltpu.VMEM((128,), jnp.int32),
            pltpu.VMEM((128,), jnp.int32),
            pltpu.VMEM((128,), jnp.int32),
            pltpu.VMEM((128,), jnp.int32),
            pltpu.VMEM((128,), jnp.int32),
            pltpu.VMEM((128,), jnp.int32),
            pltpu.VMEM((16,), jnp.int32),
            pltpu.VMEM((16,), jnp.int32),
            pltpu.VMEM((16,), jnp.int32),
            pltpu.SemaphoreType.DMA,
            pltpu.SemaphoreType.DMA,
        ],
    )(_sort_body)
    perm = sort_call(dst, src)

    gather_call = functools.partial(
        pl.kernel,
        out_type=(jax.ShapeDtypeStruct((E, EC), jnp.float32),
                  jax.ShapeDtypeStruct((E, EC), jnp.float32),
                  jax.ShapeDtypeStruct((E, EC), jnp.float32)),
        mesh=mesh,
        compiler_params=sc_params,
        scratch_types=[
            pltpu.VMEM((_GOCH,), jnp.int32),
            pltpu.VMEM((_GOCH,), jnp.int32),
            pltpu.VMEM((_GOCH, EC), jnp.float32),
            pltpu.VMEM((_GOCH, EC), jnp.float32),
            pltpu.VMEM((_GOCH, EC), jnp.float32),
            pltpu.SemaphoreType.DMA,
            pltpu.SemaphoreType.DMA,
            pltpu.SemaphoreType.DMA,
        ],
    )(_gather_body)
    tg, ej, ea2 = gather_call(t_tab, edge_attr, dst, perm)

    P = 8 * EC
    R = E // 8
    eye8 = jnp.eye(8, dtype=jnp.float32)
    wa_k = jnp.kron(eye8, WA)
    bw_k = jnp.kron(eye8, B_W.T)
    ew_k = jnp.kron(eye8, E_W.T)
    cons = jnp.stack([
        jnp.tile(B_b, 8), jnp.tile(E_b, 8),
        jnp.tile(scale, 8), jnp.tile(shift, 8)])

    BLK = 4000
    grid = (R // BLK,)
    row_spec = pl.BlockSpec((BLK, P), lambda i: (i, 0))
    full_spec = pl.BlockSpec((P, P), lambda i: (0, 0))
    out = pl.pallas_call(
        _dense_body,
        grid=grid,
        in_specs=[row_spec, row_spec, row_spec, full_spec, full_spec,
                  full_spec, pl.BlockSpec((4, P), lambda i: (0, 0))],
        out_specs=row_spec,
        out_shape=jax.ShapeDtypeStruct((R, P), jnp.float32),
    )(ea2.reshape(R, P), tg.reshape(R, P), ej.reshape(R, P),
      wa_k, bw_k, ew_k, cons)

    return out.reshape(E, EC)
